# baseline (speedup 1.0000x reference)
import functools

import jax
import jax.numpy as jnp
from jax import lax
from jax.experimental import pallas as pl
from jax.experimental.pallas import tpu as pltpu
from jax.experimental.pallas import tpu_sc as plsc

T = 2048
HIDDEN = 1024
NUM_HEADS = 16
NUM_KV_HEADS = 4
HEAD_DIM = 64
NUM_EXPERTS = 64
MOE_INTER = 512
ROPE_THETA = 10000.0
EPS = 1e-6

BLK = 32
P = 4096
NB = P // BLK

_HIGH = lax.Precision.DEFAULT


def _rmsnorm_body(x_ref, w_ref, o_ref):
    x = x_ref[...]
    v = jnp.mean(x * x, axis=-1, keepdims=True)
    o_ref[...] = x * lax.rsqrt(v + EPS) * w_ref[...]


def _rmsnorm(x, w2d):
    return pl.pallas_call(
        _rmsnorm_body,
        out_shape=jax.ShapeDtypeStruct(x.shape, x.dtype),
    )(x, w2d)


_QB = 4
_QROWS = T // _QB


def _attn_body(x_ref, w1_ref, wq_ref, wk_ref, wv_ref, cos_ref, sin_ref, o_ref,
               h1_ref, k_ref, v_ref, s_ref, oa_ref):
    h = pl.program_id(0)
    qb = pl.program_id(1)
    dn = (((1,), (1,)), ((), ()))

    @pl.when((h == 0) & (qb == 0))
    def _():
        x = x_ref[...]
        mv = jnp.mean(x * x, axis=-1, keepdims=True)
        h1_ref[...] = x * lax.rsqrt(mv + EPS) * w1_ref[...]

    def rope(y, cos, sin):
        y1 = y[:, : HEAD_DIM // 2]
        y2 = y[:, HEAD_DIM // 2:]
        return jnp.concatenate([y1 * cos - y2 * sin, y2 * cos + y1 * sin], axis=1)

    @pl.when(qb == 0)
    def _():
        h1 = h1_ref[...]
        k_ref[...] = rope(lax.dot_general(h1, wk_ref[0], dn, precision=_HIGH),
                          cos_ref[...], sin_ref[...])
        v_ref[...] = lax.dot_general(h1, wv_ref[0], dn, precision=_HIGH)

    rows = pl.ds(qb * _QROWS, _QROWS)
    q = rope(lax.dot_general(h1_ref[rows, :], wq_ref[0], dn, precision=_HIGH),
             cos_ref[rows, :], sin_ref[rows, :])
    scale = HEAD_DIM ** -0.5
    ri = lax.broadcasted_iota(jnp.int32, (_QROWS, _QROWS), 0) + qb * _QROWS
    for kb in range(_QB):
        cols = slice(kb * _QROWS, (kb + 1) * _QROWS)

        @pl.when(kb <= qb)
        def _(kb=kb, cols=cols):
            sc = lax.dot_general(q, k_ref[cols, :], dn, precision=_HIGH) * scale
            ci = lax.broadcasted_iota(jnp.int32, (_QROWS, _QROWS), 1) + kb * _QROWS
            s_ref[:, cols] = jnp.where(ci <= ri, sc, -1e9)

        @pl.when(kb > qb)
        def _(cols=cols):
            s_ref[:, cols] = jnp.full((_QROWS, _QROWS), -1e9, jnp.float32)

    s = s_ref[...]
    m = jnp.max(s, axis=-1, keepdims=True)
    p = jnp.exp(s - m)
    p = p / jnp.sum(p, axis=-1, keepdims=True)
    oa_ref[...] = jnp.zeros((_QROWS, HEAD_DIM), jnp.float32)
    for kb in range(_QB):
        cols = slice(kb * _QROWS, (kb + 1) * _QROWS)

        @pl.when(kb <= qb)
        def _(cols=cols):
            oa_ref[...] += lax.dot_general(p[:, cols], v_ref[cols, :],
                                           (((1,), (0,)), ((), ())), precision=_HIGH)

    o_ref[0] = oa_ref[...]


def _attention(x, w1d, wq3, wk3, wv3, cos, sin):
    kv_map = lambda h, qb: (h // (NUM_HEADS // NUM_KV_HEADS), 0, 0)
    return pl.pallas_call(
        _attn_body,
        grid=(NUM_HEADS, _QB),
        in_specs=[
            pl.BlockSpec((T, HIDDEN), lambda h, qb: (0, 0)),
            pl.BlockSpec((1, HIDDEN), lambda h, qb: (0, 0)),
            pl.BlockSpec((1, HEAD_DIM, HIDDEN), lambda h, qb: (h, 0, 0)),
            pl.BlockSpec((1, HEAD_DIM, HIDDEN), kv_map),
            pl.BlockSpec((1, HEAD_DIM, HIDDEN), kv_map),
            pl.BlockSpec((T, HEAD_DIM // 2), lambda h, qb: (0, 0)),
            pl.BlockSpec((T, HEAD_DIM // 2), lambda h, qb: (0, 0)),
        ],
        out_specs=pl.BlockSpec((1, _QROWS, HEAD_DIM), lambda h, qb: (h, qb, 0)),
        out_shape=jax.ShapeDtypeStruct((NUM_HEADS, T, HEAD_DIM), jnp.float32),
        scratch_shapes=[
            pltpu.VMEM((T, HIDDEN), jnp.float32),
            pltpu.VMEM((T, HEAD_DIM), jnp.float32),
            pltpu.VMEM((T, HEAD_DIM), jnp.float32),
            pltpu.VMEM((_QROWS, T), jnp.float32),
            pltpu.VMEM((_QROWS, HEAD_DIM), jnp.float32),
        ],
    )(x, w1d, wq3, wk3, wv3, cos, sin)


def _post_body(o_ref, wo_ref, x0_ref, w2_ref, gw_ref, r2_ref, h2_ref, eid_ref):
    dn = (((1,), (1,)), ((), ()))
    attn = x0_ref[...]
    for h in range(NUM_HEADS):
        attn = attn + lax.dot_general(
            o_ref[h], wo_ref[h], (((1,), (0,)), ((), ())), precision=_HIGH)
    r2 = attn
    r2_ref[...] = r2
    v = jnp.mean(r2 * r2, axis=-1, keepdims=True)
    h2 = r2 * lax.rsqrt(v + EPS) * w2_ref[...]
    h2_ref[...] = h2
    logits = lax.dot_general(h2, gw_ref[...], dn, precision=_HIGH)
    m = jnp.max(logits, axis=-1, keepdims=True)
    ci = lax.broadcasted_iota(jnp.int32, logits.shape, 1)
    cand = jnp.where(logits == m, ci, NUM_EXPERTS)
    eid_ref[...] = jnp.min(cand, axis=-1, keepdims=True)


_POST_ROWS = 256


def _post_attn(o3, woh, x0, w2d, gate_w):
    return pl.pallas_call(
        _post_body,
        grid=(T // _POST_ROWS,),
        in_specs=[
            pl.BlockSpec((NUM_HEADS, _POST_ROWS, HEAD_DIM), lambda i: (0, i, 0)),
            pl.BlockSpec((NUM_HEADS, HEAD_DIM, HIDDEN), lambda i: (0, 0, 0)),
            pl.BlockSpec((_POST_ROWS, HIDDEN), lambda i: (i, 0)),
            pl.BlockSpec((1, HIDDEN), lambda i: (0, 0)),
            pl.BlockSpec((NUM_EXPERTS, HIDDEN), lambda i: (0, 0)),
        ],
        out_specs=(
            pl.BlockSpec((_POST_ROWS, HIDDEN), lambda i: (i, 0)),
            pl.BlockSpec((_POST_ROWS, HIDDEN), lambda i: (i, 0)),
            pl.BlockSpec((_POST_ROWS, 1), lambda i: (i, 0)),
        ),
        out_shape=(
            jax.ShapeDtypeStruct((T, HIDDEN), jnp.float32),
            jax.ShapeDtypeStruct((T, HIDDEN), jnp.float32),
            jax.ShapeDtypeStruct((T, 1), jnp.int32),
        ),
    )(o3, woh, x0, w2d, gate_w)


def _moe_body(blk_ref, nbu_ref, xs_ref, wg_ref, wu_ref, wd_ref, o_ref):
    del blk_ref
    b = pl.program_id(0)

    @pl.when(b < nbu_ref[0])
    def _():
        dn = (((1,), (1,)), ((), ()))
        x = xs_ref[...].astype(jnp.bfloat16)
        wg = wg_ref[0].astype(jnp.bfloat16)
        wu = wu_ref[0].astype(jnp.bfloat16)
        wd = wd_ref[0].astype(jnp.bfloat16)
        a = lax.dot_general(x, wg, dn, preferred_element_type=jnp.float32)
        u = lax.dot_general(x, wu, dn, preferred_element_type=jnp.float32)
        h = (a * (1.0 / (1.0 + jnp.exp(-a))) * u).astype(jnp.bfloat16)
        o_ref[...] = lax.dot_general(h, wd, dn, preferred_element_type=jnp.float32)


def _moe_ffn(blk_eid, nbu, xs, w_gate, w_up, w_down):
    grid_spec = pltpu.PrefetchScalarGridSpec(
        num_scalar_prefetch=2,
        grid=(NB,),
        in_specs=[
            pl.BlockSpec((BLK, HIDDEN), lambda b, blk, nbu: (b, 0)),
            pl.BlockSpec((1, MOE_INTER, HIDDEN), lambda b, blk, nbu: (blk[b], 0, 0)),
            pl.BlockSpec((1, MOE_INTER, HIDDEN), lambda b, blk, nbu: (blk[b], 0, 0)),
            pl.BlockSpec((1, HIDDEN, MOE_INTER), lambda b, blk, nbu: (blk[b], 0, 0)),
        ],
        out_specs=pl.BlockSpec((BLK, HIDDEN), lambda b, blk, nbu: (b, 0)),
    )
    return pl.pallas_call(
        _moe_body,
        grid_spec=grid_spec,
        out_shape=jax.ShapeDtypeStruct((P, HIDDEN), jnp.float32),
    )(blk_eid, nbu, xs, w_gate, w_up, w_down)


_SC_CH = 32


def _sc_gather(table, idx3):
    nw, k, ch = idx3.shape
    n = nw * k * ch
    d = table.shape[1]
    info = plsc.get_sparse_core_info()
    mesh = plsc.VectorSubcoreMesh(core_axis_name="c", subcore_axis_name="s")

    @functools.partial(
        pl.kernel,
        mesh=mesh,
        out_type=jax.ShapeDtypeStruct((n, d), table.dtype),
        scratch_types=[
            pltpu.VMEM((ch,), jnp.int32),
            pltpu.VMEM((ch, d), table.dtype),
            pltpu.SemaphoreType.DMA,
        ],
    )
    def gk(table_hbm, idx_hbm, out_hbm, idx_v, rows_v, sem):
        wid = lax.axis_index("s") * info.num_cores + lax.axis_index("c")
        base = wid * (k * ch)
        for j in range(k):
            pltpu.sync_copy(idx_hbm.at[wid, j], idx_v)
            pltpu.async_copy(table_hbm.at[idx_v], rows_v, sem).wait()
            pltpu.sync_copy(rows_v, out_hbm.at[pl.ds(base + j * ch, ch)])

    return gk(table, idx3)


def _sc_scatter(rows, idx3, n_out):
    nw, k, ch = idx3.shape
    d = rows.shape[1]
    info = plsc.get_sparse_core_info()
    mesh = plsc.VectorSubcoreMesh(core_axis_name="c", subcore_axis_name="s")

    @functools.partial(
        pl.kernel,
        mesh=mesh,
        out_type=jax.ShapeDtypeStruct((n_out, d), rows.dtype),
        scratch_types=[
            pltpu.VMEM((ch,), jnp.int32),
            pltpu.VMEM((ch, d), rows.dtype),
            pltpu.SemaphoreType.DMA,
        ],
    )
    def sk(rows_hbm, idx_hbm, out_hbm, idx_v, rows_v, sem):
        wid = lax.axis_index("s") * info.num_cores + lax.axis_index("c")
        base = wid * (k * ch)
        for j in range(k):
            pltpu.sync_copy(idx_hbm.at[wid, j], idx_v)
            pltpu.sync_copy(rows_hbm.at[pl.ds(base + j * ch, ch)], rows_v)
            pltpu.async_copy(rows_v, out_hbm.at[idx_v], sem).wait()

    return sk(rows, idx3)


def _add_body(a_ref, b_ref, o_ref):
    o_ref[...] = a_ref[...] + b_ref[...]


def _residual_add(a, b):
    return pl.pallas_call(
        _add_body,
        out_shape=jax.ShapeDtypeStruct(a.shape, a.dtype),
    )(a, b)


def kernel(positions, hidden_states, ln1_w, wqkv, wo, ln2_w, gate_w, w_gate, w_up, w_down):
    x = hidden_states
    inv_freq = 1.0 / (ROPE_THETA ** (jnp.arange(0, HEAD_DIM, 2, dtype=jnp.float32) / HEAD_DIM))
    freqs = positions.astype(jnp.float32)[:, None] * inv_freq[None, :]
    cos = jnp.cos(freqs)
    sin = jnp.sin(freqs)

    q_size = NUM_HEADS * HEAD_DIM
    kv_size = NUM_KV_HEADS * HEAD_DIM
    wq3 = wqkv[:q_size].reshape(NUM_HEADS, HEAD_DIM, HIDDEN)
    wk3 = wqkv[q_size:q_size + kv_size].reshape(NUM_KV_HEADS, HEAD_DIM, HIDDEN)
    wv3 = wqkv[q_size + kv_size:].reshape(NUM_KV_HEADS, HEAD_DIM, HIDDEN)

    woh = wo.reshape(HIDDEN, NUM_HEADS, HEAD_DIM).transpose(1, 2, 0)

    o3 = _attention(x, ln1_w.reshape(1, HIDDEN), wq3, wk3, wv3, cos, sin)
    r2, h2, eid2 = _post_attn(o3, woh, x, ln2_w.reshape(1, HIDDEN), gate_w)

    eid = eid2[:, 0]
    order = jnp.argsort(eid).astype(jnp.int32)
    sorted_eid = eid[order]
    counts = jnp.bincount(eid, length=NUM_EXPERTS).astype(jnp.int32)
    blocks = (counts + BLK - 1) // BLK
    pad_start = (jnp.cumsum(blocks) - blocks).astype(jnp.int32) * BLK
    seg_start = (jnp.cumsum(counts) - counts).astype(jnp.int32)
    ranks = jnp.arange(T, dtype=jnp.int32) - seg_start[sorted_eid]
    dest = pad_start[sorted_eid] + ranks
    blk_eid = jnp.full((NB,), sorted_eid[-1], jnp.int32).at[dest // BLK].set(sorted_eid)
    nbu = jnp.sum(blocks, dtype=jnp.int32).reshape(1)
    gidx = jnp.zeros((T,), jnp.int32).at[order].set(dest)
    gidx3 = gidx.reshape(32, -1, _SC_CH)

    xs = _sc_scatter(h2, gidx3, P)
    ys = _moe_ffn(blk_eid, nbu, xs, w_gate, w_up, w_down)
    moe_out = _sc_gather(ys, gidx3)
    return _residual_add(r2, moe_out)


# speedup vs baseline: 1.0900x; 1.0900x over previous
import functools

import jax
import jax.numpy as jnp
from jax import lax
from jax.experimental import pallas as pl
from jax.experimental.pallas import tpu as pltpu
from jax.experimental.pallas import tpu_sc as plsc

T = 2048
HIDDEN = 1024
NUM_HEADS = 16
NUM_KV_HEADS = 4
HEAD_DIM = 64
NUM_EXPERTS = 64
MOE_INTER = 512
ROPE_THETA = 10000.0
EPS = 1e-6

BLK = 32
P = 4096
NB = P // BLK

_HIGH = lax.Precision.DEFAULT


def _attn_body(x_ref, w1_ref, wq_ref, wk_ref, wv_ref, cos_ref, sin_ref, o_ref, h1_ref):
    @pl.when(pl.program_id(0) == 0)
    def _():
        x = x_ref[...]
        v = jnp.mean(x * x, axis=-1, keepdims=True)
        h1_ref[...] = x * lax.rsqrt(v + EPS) * w1_ref[...]

    h1 = h1_ref[...]
    cos = cos_ref[...]
    sin = sin_ref[...]

    def rope(x):
        x1 = x[:, : HEAD_DIM // 2]
        x2 = x[:, HEAD_DIM // 2:]
        return jnp.concatenate([x1 * cos - x2 * sin, x2 * cos + x1 * sin], axis=1)

    dn = (((1,), (1,)), ((), ()))
    q = rope(lax.dot_general(h1, wq_ref[0], dn, precision=_HIGH))
    k = rope(lax.dot_general(h1, wk_ref[0], dn, precision=_HIGH))
    v = lax.dot_general(h1, wv_ref[0], dn, precision=_HIGH)

    s = lax.dot_general(q, k, dn, precision=_HIGH) * (HEAD_DIM ** -0.5)
    ri = lax.broadcasted_iota(jnp.int32, (T, T), 0)
    ci = lax.broadcasted_iota(jnp.int32, (T, T), 1)
    s = jnp.where(ci <= ri, s, -1e9)
    m = jnp.max(s, axis=-1, keepdims=True)
    p = jnp.exp(s - m)
    p = p / jnp.sum(p, axis=-1, keepdims=True)
    o_ref[0] = lax.dot_general(p, v, (((1,), (0,)), ((), ())), precision=_HIGH)


def _attention(x, w1d, wq3, wk3, wv3, cos, sin):
    return pl.pallas_call(
        _attn_body,
        grid=(NUM_HEADS,),
        in_specs=[
            pl.BlockSpec((T, HIDDEN), lambda h: (0, 0)),
            pl.BlockSpec((1, HIDDEN), lambda h: (0, 0)),
            pl.BlockSpec((1, HEAD_DIM, HIDDEN), lambda h: (h, 0, 0)),
            pl.BlockSpec((1, HEAD_DIM, HIDDEN), lambda h: (h // (NUM_HEADS // NUM_KV_HEADS), 0, 0)),
            pl.BlockSpec((1, HEAD_DIM, HIDDEN), lambda h: (h // (NUM_HEADS // NUM_KV_HEADS), 0, 0)),
            pl.BlockSpec((T, HEAD_DIM // 2), lambda h: (0, 0)),
            pl.BlockSpec((T, HEAD_DIM // 2), lambda h: (0, 0)),
        ],
        out_specs=pl.BlockSpec((1, T, HEAD_DIM), lambda h: (h, 0, 0)),
        out_shape=jax.ShapeDtypeStruct((NUM_HEADS, T, HEAD_DIM), jnp.float32),
        scratch_shapes=[pltpu.VMEM((T, HIDDEN), jnp.float32)],
    )(x, w1d, wq3, wk3, wv3, cos, sin)


def _post_body(o_ref, wo_ref, x0_ref, w2_ref, gw_ref, r2_ref, h2_ref, eid_ref):
    dn = (((1,), (1,)), ((), ()))
    attn = x0_ref[...]
    for h in range(NUM_HEADS):
        attn = attn + lax.dot_general(
            o_ref[h], wo_ref[h], (((1,), (0,)), ((), ())), precision=_HIGH)
    r2 = attn
    r2_ref[...] = r2
    v = jnp.mean(r2 * r2, axis=-1, keepdims=True)
    h2 = r2 * lax.rsqrt(v + EPS) * w2_ref[...]
    h2_ref[...] = h2
    logits = lax.dot_general(h2, gw_ref[...], dn, precision=_HIGH)
    m = jnp.max(logits, axis=-1, keepdims=True)
    ci = lax.broadcasted_iota(jnp.int32, logits.shape, 1)
    cand = jnp.where(logits == m, ci, NUM_EXPERTS)
    eid_ref[...] = jnp.min(cand, axis=-1, keepdims=True)


_POST_ROWS = 256


def _post_attn(o3, woh, x0, w2d, gate_w):
    return pl.pallas_call(
        _post_body,
        grid=(T // _POST_ROWS,),
        in_specs=[
            pl.BlockSpec((NUM_HEADS, _POST_ROWS, HEAD_DIM), lambda i: (0, i, 0)),
            pl.BlockSpec((NUM_HEADS, HEAD_DIM, HIDDEN), lambda i: (0, 0, 0)),
            pl.BlockSpec((_POST_ROWS, HIDDEN), lambda i: (i, 0)),
            pl.BlockSpec((1, HIDDEN), lambda i: (0, 0)),
            pl.BlockSpec((NUM_EXPERTS, HIDDEN), lambda i: (0, 0)),
        ],
        out_specs=(
            pl.BlockSpec((_POST_ROWS, HIDDEN), lambda i: (i, 0)),
            pl.BlockSpec((_POST_ROWS, HIDDEN), lambda i: (i, 0)),
            pl.BlockSpec((_POST_ROWS, 1), lambda i: (i, 0)),
        ),
        out_shape=(
            jax.ShapeDtypeStruct((T, HIDDEN), jnp.float32),
            jax.ShapeDtypeStruct((T, HIDDEN), jnp.float32),
            jax.ShapeDtypeStruct((T, 1), jnp.int32),
        ),
    )(o3, woh, x0, w2d, gate_w)


def _moe_body(blk_ref, nbu_ref, xs_ref, wg_ref, wu_ref, wd_ref, o_ref):
    del blk_ref
    b = pl.program_id(0)

    @pl.when(b < nbu_ref[0])
    def _():
        dn = (((1,), (1,)), ((), ()))
        x = xs_ref[...].astype(jnp.bfloat16)
        wg = wg_ref[0].astype(jnp.bfloat16)
        wu = wu_ref[0].astype(jnp.bfloat16)
        wd = wd_ref[0].astype(jnp.bfloat16)
        a = lax.dot_general(x, wg, dn, preferred_element_type=jnp.float32)
        u = lax.dot_general(x, wu, dn, preferred_element_type=jnp.float32)
        h = (a * (1.0 / (1.0 + jnp.exp(-a))) * u).astype(jnp.bfloat16)
        o_ref[...] = lax.dot_general(h, wd, dn, preferred_element_type=jnp.float32)


def _moe_ffn(blk_eid, nbu, xs, w_gate, w_up, w_down):
    grid_spec = pltpu.PrefetchScalarGridSpec(
        num_scalar_prefetch=2,
        grid=(NB,),
        in_specs=[
            pl.BlockSpec((BLK, HIDDEN), lambda b, blk, nbu: (b, 0)),
            pl.BlockSpec((1, MOE_INTER, HIDDEN), lambda b, blk, nbu: (blk[b], 0, 0)),
            pl.BlockSpec((1, MOE_INTER, HIDDEN), lambda b, blk, nbu: (blk[b], 0, 0)),
            pl.BlockSpec((1, HIDDEN, MOE_INTER), lambda b, blk, nbu: (blk[b], 0, 0)),
        ],
        out_specs=pl.BlockSpec((BLK, HIDDEN), lambda b, blk, nbu: (b, 0)),
    )
    return pl.pallas_call(
        _moe_body,
        grid_spec=grid_spec,
        out_shape=jax.ShapeDtypeStruct((P, HIDDEN), jnp.float32),
    )(blk_eid, nbu, xs, w_gate, w_up, w_down)


_SC_CH = 32


def _sc_gather(table, idx3):
    nw, k, ch = idx3.shape
    n = nw * k * ch
    d = table.shape[1]
    info = plsc.get_sparse_core_info()
    mesh = plsc.VectorSubcoreMesh(core_axis_name="c", subcore_axis_name="s")

    @functools.partial(
        pl.kernel,
        mesh=mesh,
        out_type=jax.ShapeDtypeStruct((n, d), table.dtype),
        scratch_types=[
            pltpu.VMEM((ch,), jnp.int32),
            pltpu.VMEM((ch, d), table.dtype),
            pltpu.SemaphoreType.DMA,
        ],
    )
    def gk(table_hbm, idx_hbm, out_hbm, idx_v, rows_v, sem):
        wid = lax.axis_index("s") * info.num_cores + lax.axis_index("c")
        base = wid * (k * ch)
        for j in range(k):
            pltpu.sync_copy(idx_hbm.at[wid, j], idx_v)
            pltpu.async_copy(table_hbm.at[idx_v], rows_v, sem).wait()
            pltpu.sync_copy(rows_v, out_hbm.at[pl.ds(base + j * ch, ch)])

    return gk(table, idx3)


def _sc_scatter(rows, idx3, n_out):
    nw, k, ch = idx3.shape
    d = rows.shape[1]
    info = plsc.get_sparse_core_info()
    mesh = plsc.VectorSubcoreMesh(core_axis_name="c", subcore_axis_name="s")

    @functools.partial(
        pl.kernel,
        mesh=mesh,
        out_type=jax.ShapeDtypeStruct((n_out, d), rows.dtype),
        scratch_types=[
            pltpu.VMEM((ch,), jnp.int32),
            pltpu.VMEM((ch, d), rows.dtype),
            pltpu.SemaphoreType.DMA,
        ],
    )
    def sk(rows_hbm, idx_hbm, out_hbm, idx_v, rows_v, sem):
        wid = lax.axis_index("s") * info.num_cores + lax.axis_index("c")
        base = wid * (k * ch)
        for j in range(k):
            pltpu.sync_copy(idx_hbm.at[wid, j], idx_v)
            pltpu.sync_copy(rows_hbm.at[pl.ds(base + j * ch, ch)], rows_v)
            pltpu.async_copy(rows_v, out_hbm.at[idx_v], sem).wait()

    return sk(rows, idx3)


def _add_body(a_ref, b_ref, o_ref):
    o_ref[...] = a_ref[...] + b_ref[...]


def _residual_add(a, b):
    return pl.pallas_call(
        _add_body,
        out_shape=jax.ShapeDtypeStruct(a.shape, a.dtype),
    )(a, b)


def kernel(positions, hidden_states, ln1_w, wqkv, wo, ln2_w, gate_w, w_gate, w_up, w_down):
    x = hidden_states
    inv_freq = 1.0 / (ROPE_THETA ** (jnp.arange(0, HEAD_DIM, 2, dtype=jnp.float32) / HEAD_DIM))
    freqs = positions.astype(jnp.float32)[:, None] * inv_freq[None, :]
    cos = jnp.cos(freqs)
    sin = jnp.sin(freqs)

    q_size = NUM_HEADS * HEAD_DIM
    kv_size = NUM_KV_HEADS * HEAD_DIM
    wq3 = wqkv[:q_size].reshape(NUM_HEADS, HEAD_DIM, HIDDEN)
    wk3 = wqkv[q_size:q_size + kv_size].reshape(NUM_KV_HEADS, HEAD_DIM, HIDDEN)
    wv3 = wqkv[q_size + kv_size:].reshape(NUM_KV_HEADS, HEAD_DIM, HIDDEN)

    woh = wo.reshape(HIDDEN, NUM_HEADS, HEAD_DIM).transpose(1, 2, 0)

    o3 = _attention(x, ln1_w.reshape(1, HIDDEN), wq3, wk3, wv3, cos, sin)
    r2, h2, eid2 = _post_attn(o3, woh, x, ln2_w.reshape(1, HIDDEN), gate_w)

    eid = eid2[:, 0]
    order = jnp.argsort(eid).astype(jnp.int32)
    sorted_eid = eid[order]
    counts = jnp.bincount(eid, length=NUM_EXPERTS).astype(jnp.int32)
    blocks = (counts + BLK - 1) // BLK
    pad_start = (jnp.cumsum(blocks) - blocks).astype(jnp.int32) * BLK
    seg_start = (jnp.cumsum(counts) - counts).astype(jnp.int32)
    ranks = jnp.arange(T, dtype=jnp.int32) - seg_start[sorted_eid]
    dest = pad_start[sorted_eid] + ranks
    blk_eid = jnp.full((NB,), sorted_eid[-1], jnp.int32).at[dest // BLK].set(sorted_eid)
    nbu = jnp.sum(blocks, dtype=jnp.int32).reshape(1)
    gidx = jnp.zeros((T,), jnp.int32).at[order].set(dest)
    gidx3 = gidx.reshape(32, -1, _SC_CH)

    xs = _sc_scatter(h2, gidx3, P)
    ys = _moe_ffn(blk_eid, nbu, xs, w_gate, w_up, w_down)
    moe_out = _sc_gather(ys, gidx3)
    return _residual_add(r2, moe_out)


# speedup vs baseline: 1.1879x; 1.0898x over previous
import functools

import jax
import jax.numpy as jnp
from jax import lax
from jax.experimental import pallas as pl
from jax.experimental.pallas import tpu as pltpu
from jax.experimental.pallas import tpu_sc as plsc

T = 2048
HIDDEN = 1024
NUM_HEADS = 16
NUM_KV_HEADS = 4
HEAD_DIM = 64
NUM_EXPERTS = 64
MOE_INTER = 512
ROPE_THETA = 10000.0
EPS = 1e-6

BLK = 64
P = 6144
NB = P // BLK

_HIGH = lax.Precision.DEFAULT


def _attn_body(x_ref, w1_ref, wq_ref, wk_ref, wv_ref, cos_ref, sin_ref, o_ref, h1_ref):
    @pl.when(pl.program_id(0) == 0)
    def _():
        x = x_ref[...]
        v = jnp.mean(x * x, axis=-1, keepdims=True)
        h1_ref[...] = x * lax.rsqrt(v + EPS) * w1_ref[...]

    h1 = h1_ref[...]
    cos = cos_ref[...]
    sin = sin_ref[...]

    def rope(x):
        x1 = x[:, : HEAD_DIM // 2]
        x2 = x[:, HEAD_DIM // 2:]
        return jnp.concatenate([x1 * cos - x2 * sin, x2 * cos + x1 * sin], axis=1)

    dn = (((1,), (1,)), ((), ()))
    q = rope(lax.dot_general(h1, wq_ref[0], dn, precision=_HIGH))
    k = rope(lax.dot_general(h1, wk_ref[0], dn, precision=_HIGH))
    v = lax.dot_general(h1, wv_ref[0], dn, precision=_HIGH)

    s = lax.dot_general(q, k, dn, precision=_HIGH) * (HEAD_DIM ** -0.5)
    ri = lax.broadcasted_iota(jnp.int32, (T, T), 0)
    ci = lax.broadcasted_iota(jnp.int32, (T, T), 1)
    s = jnp.where(ci <= ri, s, -1e9)
    m = jnp.max(s, axis=-1, keepdims=True)
    p = jnp.exp(s - m)
    p = p / jnp.sum(p, axis=-1, keepdims=True)
    o_ref[0] = lax.dot_general(p, v, (((1,), (0,)), ((), ())), precision=_HIGH)


def _attention(x, w1d, wq3, wk3, wv3, cos, sin):
    return pl.pallas_call(
        _attn_body,
        grid=(NUM_HEADS,),
        in_specs=[
            pl.BlockSpec((T, HIDDEN), lambda h: (0, 0)),
            pl.BlockSpec((1, HIDDEN), lambda h: (0, 0)),
            pl.BlockSpec((1, HEAD_DIM, HIDDEN), lambda h: (h, 0, 0)),
            pl.BlockSpec((1, HEAD_DIM, HIDDEN), lambda h: (h // (NUM_HEADS // NUM_KV_HEADS), 0, 0)),
            pl.BlockSpec((1, HEAD_DIM, HIDDEN), lambda h: (h // (NUM_HEADS // NUM_KV_HEADS), 0, 0)),
            pl.BlockSpec((T, HEAD_DIM // 2), lambda h: (0, 0)),
            pl.BlockSpec((T, HEAD_DIM // 2), lambda h: (0, 0)),
        ],
        out_specs=pl.BlockSpec((1, T, HEAD_DIM), lambda h: (h, 0, 0)),
        out_shape=jax.ShapeDtypeStruct((NUM_HEADS, T, HEAD_DIM), jnp.float32),
        scratch_shapes=[pltpu.VMEM((T, HIDDEN), jnp.float32)],
    )(x, w1d, wq3, wk3, wv3, cos, sin)


def _post_body(o_ref, wo_ref, x0_ref, w2_ref, gw_ref, r2_ref, h2_ref, eid_ref):
    dn = (((1,), (1,)), ((), ()))
    attn = x0_ref[...]
    for h in range(NUM_HEADS):
        attn = attn + lax.dot_general(
            o_ref[h], wo_ref[h], (((1,), (0,)), ((), ())), precision=_HIGH)
    r2 = attn
    r2_ref[...] = r2
    v = jnp.mean(r2 * r2, axis=-1, keepdims=True)
    h2 = r2 * lax.rsqrt(v + EPS) * w2_ref[...]
    h2_ref[...] = h2
    logits = lax.dot_general(h2, gw_ref[...], dn, precision=_HIGH)
    m = jnp.max(logits, axis=-1, keepdims=True)
    ci = lax.broadcasted_iota(jnp.int32, logits.shape, 1)
    cand = jnp.where(logits == m, ci, NUM_EXPERTS)
    eid_ref[...] = jnp.min(cand, axis=-1, keepdims=True)


_POST_ROWS = 256


def _post_attn(o3, woh, x0, w2d, gate_w):
    return pl.pallas_call(
        _post_body,
        grid=(T // _POST_ROWS,),
        in_specs=[
            pl.BlockSpec((NUM_HEADS, _POST_ROWS, HEAD_DIM), lambda i: (0, i, 0)),
            pl.BlockSpec((NUM_HEADS, HEAD_DIM, HIDDEN), lambda i: (0, 0, 0)),
            pl.BlockSpec((_POST_ROWS, HIDDEN), lambda i: (i, 0)),
            pl.BlockSpec((1, HIDDEN), lambda i: (0, 0)),
            pl.BlockSpec((NUM_EXPERTS, HIDDEN), lambda i: (0, 0)),
        ],
        out_specs=(
            pl.BlockSpec((_POST_ROWS, HIDDEN), lambda i: (i, 0)),
            pl.BlockSpec((_POST_ROWS, HIDDEN), lambda i: (i, 0)),
            pl.BlockSpec((_POST_ROWS, 1), lambda i: (i, 0)),
        ),
        out_shape=(
            jax.ShapeDtypeStruct((T, HIDDEN), jnp.float32),
            jax.ShapeDtypeStruct((T, HIDDEN), jnp.float32),
            jax.ShapeDtypeStruct((T, 1), jnp.int32),
        ),
    )(o3, woh, x0, w2d, gate_w)


def _moe_body(blk_ref, nbu_ref, xs_ref, wg_ref, wu_ref, wd_ref, o_ref):
    del blk_ref
    b = pl.program_id(0)

    @pl.when(b < nbu_ref[0])
    def _():
        dn = (((1,), (1,)), ((), ()))
        x = xs_ref[...].astype(jnp.bfloat16)
        wg = wg_ref[0].astype(jnp.bfloat16)
        wu = wu_ref[0].astype(jnp.bfloat16)
        wd = wd_ref[0].astype(jnp.bfloat16)
        a = lax.dot_general(x, wg, dn, preferred_element_type=jnp.float32)
        u = lax.dot_general(x, wu, dn, preferred_element_type=jnp.float32)
        h = (a * (1.0 / (1.0 + jnp.exp(-a))) * u).astype(jnp.bfloat16)
        o_ref[...] = lax.dot_general(h, wd, dn, preferred_element_type=jnp.float32)


def _moe_ffn(blk_eid, nbu, xs, w_gate, w_up, w_down):
    grid_spec = pltpu.PrefetchScalarGridSpec(
        num_scalar_prefetch=2,
        grid=(NB,),
        in_specs=[
            pl.BlockSpec((BLK, HIDDEN), lambda b, blk, nbu: (b, 0)),
            pl.BlockSpec((1, MOE_INTER, HIDDEN), lambda b, blk, nbu: (blk[b], 0, 0)),
            pl.BlockSpec((1, MOE_INTER, HIDDEN), lambda b, blk, nbu: (blk[b], 0, 0)),
            pl.BlockSpec((1, HIDDEN, MOE_INTER), lambda b, blk, nbu: (blk[b], 0, 0)),
        ],
        out_specs=pl.BlockSpec((BLK, HIDDEN), lambda b, blk, nbu: (b, 0)),
    )
    return pl.pallas_call(
        _moe_body,
        grid_spec=grid_spec,
        out_shape=jax.ShapeDtypeStruct((P, HIDDEN), jnp.float32),
    )(blk_eid, nbu, xs, w_gate, w_up, w_down)


_SC_CH = 32


def _sc_gather(table, idx3):
    nw, k, ch = idx3.shape
    n = nw * k * ch
    d = table.shape[1]
    info = plsc.get_sparse_core_info()
    mesh = plsc.VectorSubcoreMesh(core_axis_name="c", subcore_axis_name="s")

    @functools.partial(
        pl.kernel,
        mesh=mesh,
        out_type=jax.ShapeDtypeStruct((n, d), table.dtype),
        scratch_types=[
            pltpu.VMEM((ch,), jnp.int32),
            pltpu.VMEM((ch, d), table.dtype),
            pltpu.SemaphoreType.DMA,
        ],
    )
    def gk(table_hbm, idx_hbm, out_hbm, idx_v, rows_v, sem):
        wid = lax.axis_index("s") * info.num_cores + lax.axis_index("c")
        base = wid * (k * ch)
        for j in range(k):
            pltpu.sync_copy(idx_hbm.at[wid, j], idx_v)
            pltpu.async_copy(table_hbm.at[idx_v], rows_v, sem).wait()
            pltpu.sync_copy(rows_v, out_hbm.at[pl.ds(base + j * ch, ch)])

    return gk(table, idx3)


def _sc_scatter(rows, idx3, n_out):
    nw, k, ch = idx3.shape
    d = rows.shape[1]
    info = plsc.get_sparse_core_info()
    mesh = plsc.VectorSubcoreMesh(core_axis_name="c", subcore_axis_name="s")

    @functools.partial(
        pl.kernel,
        mesh=mesh,
        out_type=jax.ShapeDtypeStruct((n_out, d), rows.dtype),
        scratch_types=[
            pltpu.VMEM((ch,), jnp.int32),
            pltpu.VMEM((ch, d), rows.dtype),
            pltpu.SemaphoreType.DMA,
        ],
    )
    def sk(rows_hbm, idx_hbm, out_hbm, idx_v, rows_v, sem):
        wid = lax.axis_index("s") * info.num_cores + lax.axis_index("c")
        base = wid * (k * ch)
        for j in range(k):
            pltpu.sync_copy(idx_hbm.at[wid, j], idx_v)
            pltpu.sync_copy(rows_hbm.at[pl.ds(base + j * ch, ch)], rows_v)
            pltpu.async_copy(rows_v, out_hbm.at[idx_v], sem).wait()

    return sk(rows, idx3)


def _add_body(a_ref, b_ref, o_ref):
    o_ref[...] = a_ref[...] + b_ref[...]


def _residual_add(a, b):
    return pl.pallas_call(
        _add_body,
        out_shape=jax.ShapeDtypeStruct(a.shape, a.dtype),
    )(a, b)


def kernel(positions, hidden_states, ln1_w, wqkv, wo, ln2_w, gate_w, w_gate, w_up, w_down):
    x = hidden_states
    inv_freq = 1.0 / (ROPE_THETA ** (jnp.arange(0, HEAD_DIM, 2, dtype=jnp.float32) / HEAD_DIM))
    freqs = positions.astype(jnp.float32)[:, None] * inv_freq[None, :]
    cos = jnp.cos(freqs)
    sin = jnp.sin(freqs)

    q_size = NUM_HEADS * HEAD_DIM
    kv_size = NUM_KV_HEADS * HEAD_DIM
    wq3 = wqkv[:q_size].reshape(NUM_HEADS, HEAD_DIM, HIDDEN)
    wk3 = wqkv[q_size:q_size + kv_size].reshape(NUM_KV_HEADS, HEAD_DIM, HIDDEN)
    wv3 = wqkv[q_size + kv_size:].reshape(NUM_KV_HEADS, HEAD_DIM, HIDDEN)

    woh = wo.reshape(HIDDEN, NUM_HEADS, HEAD_DIM).transpose(1, 2, 0)

    o3 = _attention(x, ln1_w.reshape(1, HIDDEN), wq3, wk3, wv3, cos, sin)
    r2, h2, eid2 = _post_attn(o3, woh, x, ln2_w.reshape(1, HIDDEN), gate_w)

    eid = eid2[:, 0]
    order = jnp.argsort(eid).astype(jnp.int32)
    sorted_eid = eid[order]
    counts = jnp.bincount(eid, length=NUM_EXPERTS).astype(jnp.int32)
    blocks = (counts + BLK - 1) // BLK
    pad_start = (jnp.cumsum(blocks) - blocks).astype(jnp.int32) * BLK
    seg_start = (jnp.cumsum(counts) - counts).astype(jnp.int32)
    ranks = jnp.arange(T, dtype=jnp.int32) - seg_start[sorted_eid]
    dest = pad_start[sorted_eid] + ranks
    blk_eid = jnp.full((NB,), sorted_eid[-1], jnp.int32).at[dest // BLK].set(sorted_eid)
    nbu = jnp.sum(blocks, dtype=jnp.int32).reshape(1)
    gidx = jnp.zeros((T,), jnp.int32).at[order].set(dest)
    gidx3 = gidx.reshape(32, -1, _SC_CH)

    xs = _sc_scatter(h2, gidx3, P)
    ys = _moe_ffn(blk_eid, nbu, xs, w_gate, w_up, w_down)
    moe_out = _sc_gather(ys, gidx3)
    return _residual_add(r2, moe_out)
